# baseline (speedup 1.0000x reference)
import functools

import jax
import jax.numpy as jnp
from jax import lax
from jax.experimental import pallas as pl
from jax.experimental.pallas import tpu as pltpu
from jax.experimental.pallas import tpu_sc as plsc

B, N, NPOINT = 4, 8192, 1024
RADII = (0.1, 0.2)
NSAMPLES = (32, 64)
IN_CH = 64
OUT = 128


def _fps_kernel(xyz_ref, o_ref):
    rows, cols = 8, N // 8
    idx2d = (jax.lax.broadcasted_iota(jnp.int32, (rows, cols), 0) * cols
             + jax.lax.broadcasted_iota(jnp.int32, (rows, cols), 1))
    ocols = NPOINT // 8
    t2d = (jax.lax.broadcasted_iota(jnp.int32, (8, ocols), 0) * ocols
           + jax.lax.broadcasted_iota(jnp.int32, (8, ocols), 1))

    def extract(sel2, arr):
        return jnp.max(jnp.where(sel2, arr, -1.0))

    init = []
    sel0 = idx2d == 0
    hit0 = t2d == 0
    zeros = jnp.zeros((8, ocols), jnp.float32)
    for b in range(B):
        cx = extract(sel0, xyz_ref[b, 0])
        cy = extract(sel0, xyz_ref[b, 1])
        cz = extract(sel0, xyz_ref[b, 2])
        init.append((jnp.full((rows, cols), 1e10, jnp.float32), cx, cy, cz,
                     jnp.where(hit0, cx, zeros), jnp.where(hit0, cy, zeros),
                     jnp.where(hit0, cz, zeros)))

    def body(t, carry):
        hit = t2d == t
        out = []
        for b in range(B):
            dists, cx, cy, cz, newx, newy, newz = carry[b]
            x = xyz_ref[b, 0]
            y = xyz_ref[b, 1]
            z = xyz_ref[b, 2]
            dx = x - cx
            dy = y - cy
            dz = z - cz
            d = dx * dx + dy * dy + dz * dz
            dists = jnp.minimum(dists, d)
            maxv = jnp.max(dists)
            fidx = jnp.min(jnp.where(dists == maxv, idx2d, N))
            sel2 = idx2d == fidx
            cx = extract(sel2, x)
            cy = extract(sel2, y)
            cz = extract(sel2, z)
            out.append((dists, cx, cy, cz, jnp.where(hit, cx, newx),
                        jnp.where(hit, cy, newy), jnp.where(hit, cz, newz)))
        return tuple(out)

    carry = jax.lax.fori_loop(1, NPOINT, body, tuple(init))
    for b in range(B):
        _, _, _, _, newx, newy, newz = carry[b]
        o_ref[b, 0] = newx
        o_ref[b, 1] = newy
        o_ref[b, 2] = newz


def _fps_newxyz(xyz):
    b, n, _ = xyz.shape
    xyzT = jnp.transpose(xyz, (0, 2, 1))
    xyzR = xyzT.reshape(b, 3, 8, n // 8)
    outR = pl.pallas_call(
        _fps_kernel,
        out_shape=jax.ShapeDtypeStruct((b, 3, 8, NPOINT // 8), jnp.float32),
    )(xyzR)
    cxyzT = outR.reshape(b, 3, NPOINT)
    return jnp.transpose(cxyzT, (0, 2, 1)), cxyzT, xyzT


_NW = 32
_MW = NPOINT // 8
_GROWS = 256


def _sc_body(xyzT_h, cxyzT_h, feat_h, gf0_h, gf1_h, rel0_h, rel1_h,
             x_v, y_v, z_v, cx_v, cy_v, cz_v, i0_v, i1_v, g0_v, g1_v,
             r0_v, r1_v, gf_v, sem):
    K0, K1 = NSAMPLES
    wid = lax.axis_index("s") * 2 + lax.axis_index("c")
    b = wid // 8
    m0 = (wid % 8) * _MW
    pltpu.sync_copy(xyzT_h.at[pl.ds((b * 3 + 0) * N, N)], x_v)
    pltpu.sync_copy(xyzT_h.at[pl.ds((b * 3 + 1) * N, N)], y_v)
    pltpu.sync_copy(xyzT_h.at[pl.ds((b * 3 + 2) * N, N)], z_v)
    pltpu.sync_copy(cxyzT_h.at[pl.ds((b * 3 + 0) * NPOINT + m0, _MW)], cx_v)
    pltpu.sync_copy(cxyzT_h.at[pl.ds((b * 3 + 1) * NPOINT + m0, _MW)], cy_v)
    pltpu.sync_copy(cxyzT_h.at[pl.ds((b * 3 + 2) * NPOINT + m0, _MW)], cz_v)
    iota = lax.iota(jnp.int32, 16)
    r2a = jnp.float32(RADII[0] * RADII[0])
    r2b = jnp.float32(RADII[1] * RADII[1])
    nchunk = N // 16
    boff = b * N

    nsup = nchunk // 16

    def per_centroid(m, carry_unused):
        mm = jnp.full((16,), m, jnp.int32)
        cxv = plsc.load_gather(cx_v, [mm])
        cyv = plsc.load_gather(cy_v, [mm])
        czv = plsc.load_gather(cz_v, [mm])
        base0 = m * K0
        base1 = m * K1
        zc = jnp.zeros((16,), jnp.int32)

        def scan_body(j, cnts):
            c0v, c1v = cnts
            off = j * 16
            dx = x_v[pl.ds(off, 16)] - cxv
            dy = y_v[pl.ds(off, 16)] - cyv
            dz = z_v[pl.ds(off, 16)] - czv
            d2 = dx * dx + dy * dy + dz * dz
            mk0 = d2 < r2a
            mk1 = d2 < r2b
            idxv = iota + off
            rk0 = plsc.cumsum(mk0.astype(jnp.int32))
            rk1 = plsc.cumsum(mk1.astype(jnp.int32))
            pos0 = jnp.minimum(c0v + (rk0 - 1), K0 + 15) + base0
            pos1 = jnp.minimum(c1v + (rk1 - 1), K1 + 15) + base1
            plsc.store_scatter(i0_v, [pos0], idxv, mask=mk0)
            plsc.store_scatter(i1_v, [pos1], idxv, mask=mk1)
            c0v = c0v + plsc.all_reduce_population_count(mk0)
            c1v = c1v + plsc.all_reduce_population_count(mk1)
            return c0v, c1v

        c0v, c1v = plsc.parallel_loop(
            0, nchunk, 1, unroll=16, carry=(zc, zc))(scan_body)
        c0 = jnp.max(c0v)
        c1 = jnp.max(c1v)

        for (k, cN, base, iv, gv, rv) in (
                (K0, c0, base0, i0_v, g0_v, r0_v),
                (K1, c1, base1, i1_v, g1_v, r1_v)):
            cf = jnp.minimum(cN, k)
            fv = plsc.load_gather(iv, [jnp.full((16,), base, jnp.int32)])
            for t in range(k // 16):
                lane = iota + t * 16
                cur = iv[pl.ds(base + t * 16, 16)]
                cur = jnp.where(lane < cf, cur, fv)
                gv[pl.ds(base + t * 16, 16)] = cur + boff
                gx = plsc.load_gather(x_v, [cur]) - cxv
                gy = plsc.load_gather(y_v, [cur]) - cyv
                gz = plsc.load_gather(z_v, [cur]) - czv
                sidx = (iota * 3) + (base + t * 16) * 3
                plsc.store_scatter(rv, [sidx], gx)
                plsc.store_scatter(rv, [sidx + 1], gy)
                plsc.store_scatter(rv, [sidx + 2], gz)
        return carry_unused

    lax.fori_loop(0, _MW, per_centroid, jnp.int32(0))

    K0T, K1T = _MW * K0, _MW * K1
    for g in range(K0T // _GROWS):
        pltpu.async_copy(
            feat_h.at[g0_v.at[pl.ds(g * _GROWS, _GROWS)]], gf_v, sem).wait()
        pltpu.sync_copy(
            gf_v, gf0_h.at[pl.ds((b * NPOINT + m0) * K0 + g * _GROWS, _GROWS)])
    for g in range(K1T // _GROWS):
        pltpu.async_copy(
            feat_h.at[g1_v.at[pl.ds(g * _GROWS, _GROWS)]], gf_v, sem).wait()
        pltpu.sync_copy(
            gf_v, gf1_h.at[pl.ds((b * NPOINT + m0) * K1 + g * _GROWS, _GROWS)])
    pltpu.sync_copy(r0_v, rel0_h.at[pl.ds((b * NPOINT + m0) * K0 * 3, K0T * 3)])
    pltpu.sync_copy(r1_v, rel1_h.at[pl.ds((b * NPOINT + m0) * K1 * 3, K1T * 3)])


def _sc_ballquery_gather(xyzT, cxyzT, featflat):
    K0, K1 = NSAMPLES
    mesh = plsc.VectorSubcoreMesh(core_axis_name="c", subcore_axis_name="s")
    out_type = [
        jax.ShapeDtypeStruct((B * NPOINT * K0, IN_CH), jnp.float32),
        jax.ShapeDtypeStruct((B * NPOINT * K1, IN_CH), jnp.float32),
        jax.ShapeDtypeStruct((B * NPOINT * K0 * 3,), jnp.float32),
        jax.ShapeDtypeStruct((B * NPOINT * K1 * 3,), jnp.float32),
    ]
    scratch = [
        pltpu.VMEM((N,), jnp.float32),
        pltpu.VMEM((N,), jnp.float32),
        pltpu.VMEM((N,), jnp.float32),
        pltpu.VMEM((_MW,), jnp.float32),
        pltpu.VMEM((_MW,), jnp.float32),
        pltpu.VMEM((_MW,), jnp.float32),
        pltpu.VMEM((_MW * K0 + 16,), jnp.int32),
        pltpu.VMEM((_MW * K1 + 16,), jnp.int32),
        pltpu.VMEM((_MW * K0,), jnp.int32),
        pltpu.VMEM((_MW * K1,), jnp.int32),
        pltpu.VMEM((_MW * K0 * 3,), jnp.float32),
        pltpu.VMEM((_MW * K1 * 3,), jnp.float32),
        pltpu.VMEM((_GROWS, IN_CH), jnp.float32),
        pltpu.SemaphoreType.DMA,
    ]
    gf0, gf1, rel0, rel1 = pl.kernel(
        _sc_body, out_type=out_type, mesh=mesh, scratch_types=scratch,
        compiler_params=pltpu.CompilerParams(
            needs_layout_passes=False, use_tc_tiling_on_sc=False),
    )(xyzT.reshape(-1), cxyzT.reshape(-1), featflat)
    return gf0, gf1, rel0.reshape(-1, 3), rel1.reshape(-1, 3)


def _ball_query(radius, k, xyz, new_xyz):
    b, n, _ = xyz.shape
    d2 = jnp.sum((new_xyz[:, :, None, :] - xyz[:, None, :, :]) ** 2, axis=-1)
    mask = d2 < radius * radius
    key = jnp.where(mask, jnp.arange(n)[None, None, :], n)
    srt = jnp.sort(key, axis=-1)[..., :k]
    first = srt[..., :1]
    idx = jnp.where(srt >= n, first, srt)
    idx = jnp.where(idx >= n, 0, idx)
    return idx


def _bn(x, gamma, beta, axes):
    mu = jnp.mean(x, axis=axes, keepdims=True)
    var = jnp.var(x, axis=axes, keepdims=True)
    return (x - mu) / jnp.sqrt(var + 1e-5) * gamma + beta


_TR = 2048


def _mlp_y(gf_ref, rel_ref, wax_ref, waf_ref):
    gf = gf_ref[...]
    rel = rel_ref[...]
    y = jax.lax.dot_general(gf, waf_ref[...], (((1,), (0,)), ((), ())),
                            preferred_element_type=jnp.float32)
    y = y + rel[:, 0:1] * wax_ref[0:1, :]
    y = y + rel[:, 1:2] * wax_ref[1:2, :]
    y = y + rel[:, 2:3] * wax_ref[2:3, :]
    return y


def _p1_kernel(gf_ref, rel_ref, wax_ref, waf_ref, s_ref):
    y = _mlp_y(gf_ref, rel_ref, wax_ref, waf_ref)

    @pl.when(pl.program_id(0) == 0)
    def _():
        s_ref[...] = jnp.zeros_like(s_ref)

    s_ref[0:1, :] += jnp.sum(y, 0, keepdims=True)
    s_ref[1:2, :] += jnp.sum(y * y, 0, keepdims=True)


def _p2_kernel(gf_ref, rel_ref, wax_ref, waf_ref, a1_ref, c1_ref, wb_ref,
               s_ref):
    y = _mlp_y(gf_ref, rel_ref, wax_ref, waf_ref)
    h = jnp.maximum(y * a1_ref[...] + c1_ref[...], 0.0)
    z = jax.lax.dot_general(h, wb_ref[...], (((1,), (0,)), ((), ())),
                            preferred_element_type=jnp.float32)

    @pl.when(pl.program_id(0) == 0)
    def _():
        s_ref[...] = jnp.zeros_like(s_ref)

    s_ref[0:1, :] += jnp.sum(z, 0, keepdims=True)
    s_ref[1:2, :] += jnp.sum(z * z, 0, keepdims=True)


def _p3_kernel(k_const, gf_ref, rel_ref, wax_ref, waf_ref, a1_ref, c1_ref,
               wb_ref, a2_ref, c2_ref, o_ref):
    y = _mlp_y(gf_ref, rel_ref, wax_ref, waf_ref)
    h = jnp.maximum(y * a1_ref[...] + c1_ref[...], 0.0)
    z = jax.lax.dot_general(h, wb_ref[...], (((1,), (0,)), ((), ())),
                            preferred_element_type=jnp.float32)
    o = jnp.maximum(z * a2_ref[...] + c2_ref[...], 0.0)
    o_ref[...] = jnp.max(o.reshape(_TR // k_const, k_const, OUT), axis=1)


def _branch_mlp(gf, rel, k, Wa, ga, ba, Wb, gb, bb):
    R = gf.shape[0]
    ntiles = R // _TR
    wax = Wa[:, :3].T
    waf = Wa[:, 3:].T
    wbT = Wb.T
    half = Wa.shape[0]
    row_spec = [
        pl.BlockSpec((_TR, IN_CH), lambda i: (i, 0)),
        pl.BlockSpec((_TR, 3), lambda i: (i, 0)),
        pl.BlockSpec((3, half), lambda i: (0, 0)),
        pl.BlockSpec((IN_CH, half), lambda i: (0, 0)),
    ]
    acc_spec = pl.BlockSpec((2, half), lambda i: (0, 0))
    s1 = pl.pallas_call(
        _p1_kernel, grid=(ntiles,), in_specs=row_spec, out_specs=acc_spec,
        out_shape=jax.ShapeDtypeStruct((2, half), jnp.float32),
    )(gf, rel, wax, waf)
    mu1 = s1[0] / R
    var1 = s1[1] / R - mu1 * mu1
    a1 = (ga / jnp.sqrt(var1 + 1e-5)).reshape(1, half)
    c1 = (ba - mu1 * ga / jnp.sqrt(var1 + 1e-5)).reshape(1, half)

    vec_specs = [
        pl.BlockSpec((1, half), lambda i: (0, 0)),
        pl.BlockSpec((1, half), lambda i: (0, 0)),
        pl.BlockSpec((half, OUT), lambda i: (0, 0)),
    ]
    acc2_spec = pl.BlockSpec((2, OUT), lambda i: (0, 0))
    s2 = pl.pallas_call(
        _p2_kernel, grid=(ntiles,), in_specs=row_spec + vec_specs,
        out_specs=acc2_spec,
        out_shape=jax.ShapeDtypeStruct((2, OUT), jnp.float32),
    )(gf, rel, wax, waf, a1, c1, wbT)
    mu2 = s2[0] / R
    var2 = s2[1] / R - mu2 * mu2
    a2 = (gb / jnp.sqrt(var2 + 1e-5)).reshape(1, OUT)
    c2 = (bb - mu2 * gb / jnp.sqrt(var2 + 1e-5)).reshape(1, OUT)

    vec2_specs = [
        pl.BlockSpec((1, OUT), lambda i: (0, 0)),
        pl.BlockSpec((1, OUT), lambda i: (0, 0)),
    ]
    pooled = pl.pallas_call(
        functools.partial(_p3_kernel, k),
        grid=(ntiles,), in_specs=row_spec + vec_specs + vec2_specs,
        out_specs=pl.BlockSpec((_TR // k, OUT), lambda i: (i, 0)),
        out_shape=jax.ShapeDtypeStruct((R // k, OUT), jnp.float32),
    )(gf, rel, wax, waf, a1, c1, wbT, a2, c2)
    return pooled.reshape(B, NPOINT, OUT)


def _final_mlp_kernel(x_ref, wf1_ref, gf1_ref, bf1_ref, wf2_ref, gf2_ref, bf2_ref, o_ref):
    x = x_ref[...]
    y = jax.lax.dot_general(x, wf1_ref[...], (((1,), (1,)), ((), ())),
                            preferred_element_type=jnp.float32)
    mu = jnp.mean(y, axis=0, keepdims=True)
    var = jnp.mean((y - mu) ** 2, axis=0, keepdims=True)
    y = (y - mu) / jnp.sqrt(var + 1e-5) * gf1_ref[...] + bf1_ref[...]
    y = jnp.maximum(y, 0.0)
    z = jax.lax.dot_general(y, wf2_ref[...], (((1,), (1,)), ((), ())),
                            preferred_element_type=jnp.float32)
    mu = jnp.mean(z, axis=0, keepdims=True)
    var = jnp.mean((z - mu) ** 2, axis=0, keepdims=True)
    z = (z - mu) / jnp.sqrt(var + 1e-5) * gf2_ref[...] + bf2_ref[...]
    o_ref[...] = jnp.maximum(z, 0.0)


def _final_mlp(x, Wf1, gf1, bf1, Wf2, gf2, bf2):
    bm = x.shape[0] * x.shape[1]
    out = pl.pallas_call(
        _final_mlp_kernel,
        out_shape=jax.ShapeDtypeStruct((bm, OUT), jnp.float32),
    )(x.reshape(bm, -1), Wf1, gf1.reshape(1, OUT), bf1.reshape(1, OUT),
      Wf2, gf2.reshape(1, OUT), bf2.reshape(1, OUT))
    return out.reshape(x.shape[0], x.shape[1], OUT)


def kernel(xyz, feat, W0_1, g0_1, b0_1, W0_2, g0_2, b0_2, W1_1, g1_1, b1_1,
           W1_2, g1_2, b1_2, Wf1, gf1, bf1, Wf2, gf2, bf2):
    new_xyz, cxyzT, xyzT = _fps_newxyz(xyz)
    featflat = feat.reshape(B * N, IN_CH)
    gfa, gfb, rela, relb = _sc_ballquery_gather(xyzT, cxyzT, featflat)
    pooled0 = _branch_mlp(gfa, rela, NSAMPLES[0], W0_1, g0_1, b0_1, W0_2, g0_2, b0_2)
    pooled1 = _branch_mlp(gfb, relb, NSAMPLES[1], W1_1, g1_1, b1_1, W1_2, g1_2, b1_2)
    x = jnp.concatenate([pooled0, pooled1], axis=-1)
    x = _final_mlp(x, Wf1, gf1, bf1, Wf2, gf2, bf2)
    return (new_xyz, x)


# speedup vs baseline: 1.1768x; 1.1768x over previous
import functools

import jax
import jax.numpy as jnp
from jax import lax
from jax.experimental import pallas as pl
from jax.experimental.pallas import tpu as pltpu
from jax.experimental.pallas import tpu_sc as plsc

B, N, NPOINT = 4, 8192, 1024
RADII = (0.1, 0.2)
NSAMPLES = (32, 64)
IN_CH = 64
OUT = 128


def _fps_kernel(xyz_ref, o_ref):
    rows, cols = 8, N // 8
    idx2d = (jax.lax.broadcasted_iota(jnp.int32, (rows, cols), 0) * cols
             + jax.lax.broadcasted_iota(jnp.int32, (rows, cols), 1))
    ocols = NPOINT // 8
    t2d = (jax.lax.broadcasted_iota(jnp.int32, (8, ocols), 0) * ocols
           + jax.lax.broadcasted_iota(jnp.int32, (8, ocols), 1))

    def extract(sel2, arr):
        return jnp.max(jnp.where(sel2, arr, -1.0))

    init = []
    sel0 = idx2d == 0
    hit0 = t2d == 0
    zeros = jnp.zeros((8, ocols), jnp.float32)
    for b in range(B):
        cx = extract(sel0, xyz_ref[b, 0])
        cy = extract(sel0, xyz_ref[b, 1])
        cz = extract(sel0, xyz_ref[b, 2])
        init.append((jnp.full((rows, cols), 1e10, jnp.float32), cx, cy, cz,
                     jnp.where(hit0, cx, zeros), jnp.where(hit0, cy, zeros),
                     jnp.where(hit0, cz, zeros)))

    def body(t, carry):
        hit = t2d == t
        out = []
        for b in range(B):
            dists, cx, cy, cz, newx, newy, newz = carry[b]
            x = xyz_ref[b, 0]
            y = xyz_ref[b, 1]
            z = xyz_ref[b, 2]
            dx = x - cx
            dy = y - cy
            dz = z - cz
            d = dx * dx + dy * dy + dz * dz
            dists = jnp.minimum(dists, d)
            maxv = jnp.max(dists)
            fidx = jnp.min(jnp.where(dists == maxv, idx2d, N))
            sel2 = idx2d == fidx
            cx = extract(sel2, x)
            cy = extract(sel2, y)
            cz = extract(sel2, z)
            out.append((dists, cx, cy, cz, jnp.where(hit, cx, newx),
                        jnp.where(hit, cy, newy), jnp.where(hit, cz, newz)))
        return tuple(out)

    carry = jax.lax.fori_loop(1, NPOINT, body, tuple(init))
    for b in range(B):
        _, _, _, _, newx, newy, newz = carry[b]
        o_ref[b, 0] = newx
        o_ref[b, 1] = newy
        o_ref[b, 2] = newz


def _fps_newxyz(xyz):
    b, n, _ = xyz.shape
    xyzT = jnp.transpose(xyz, (0, 2, 1))
    xyzR = xyzT.reshape(b, 3, 8, n // 8)
    outR = pl.pallas_call(
        _fps_kernel,
        out_shape=jax.ShapeDtypeStruct((b, 3, 8, NPOINT // 8), jnp.float32),
    )(xyzR)
    cxyzT = outR.reshape(b, 3, NPOINT)
    return jnp.transpose(cxyzT, (0, 2, 1)), cxyzT, xyzT


_NW = 32
_MW = NPOINT // 8
_GROWS = 256


def _sc_body(xyzT_h, cxyzT_h, feat_h, gf0_h, gf1_h, rel0_h, rel1_h,
             x_v, y_v, z_v, cx_v, cy_v, cz_v, i0_v, i1_v, g0_v, g1_v,
             r0_v, r1_v, gf_v, sem):
    K0, K1 = NSAMPLES
    wid = lax.axis_index("s") * 2 + lax.axis_index("c")
    b = wid // 8
    m0 = (wid % 8) * _MW
    pltpu.sync_copy(xyzT_h.at[pl.ds((b * 3 + 0) * N, N)], x_v)
    pltpu.sync_copy(xyzT_h.at[pl.ds((b * 3 + 1) * N, N)], y_v)
    pltpu.sync_copy(xyzT_h.at[pl.ds((b * 3 + 2) * N, N)], z_v)
    pltpu.sync_copy(cxyzT_h.at[pl.ds((b * 3 + 0) * NPOINT + m0, _MW)], cx_v)
    pltpu.sync_copy(cxyzT_h.at[pl.ds((b * 3 + 1) * NPOINT + m0, _MW)], cy_v)
    pltpu.sync_copy(cxyzT_h.at[pl.ds((b * 3 + 2) * NPOINT + m0, _MW)], cz_v)
    iota = lax.iota(jnp.int32, 16)
    r2a = jnp.float32(RADII[0] * RADII[0])
    r2b = jnp.float32(RADII[1] * RADII[1])
    nchunk = N // 16
    boff = b * N

    nsup = nchunk // 16

    def per_centroid(m, carry_unused):
        mm = jnp.full((16,), m, jnp.int32)
        cxv = plsc.load_gather(cx_v, [mm])
        cyv = plsc.load_gather(cy_v, [mm])
        czv = plsc.load_gather(cz_v, [mm])
        base0 = m * K0
        base1 = m * K1
        zc = jnp.zeros((16,), jnp.int32)

        def scan_body(j, cnts):
            c0v, c1v = cnts
            off = j * 16
            dx = x_v[pl.ds(off, 16)] - cxv
            dy = y_v[pl.ds(off, 16)] - cyv
            dz = z_v[pl.ds(off, 16)] - czv
            d2 = dx * dx + dy * dy + dz * dz
            mk0 = d2 < r2a
            mk1 = d2 < r2b
            idxv = iota + off
            rk0 = plsc.cumsum(mk0.astype(jnp.int32))
            rk1 = plsc.cumsum(mk1.astype(jnp.int32))
            pos0 = jnp.minimum(c0v + (rk0 - 1), K0 + 15) + base0
            pos1 = jnp.minimum(c1v + (rk1 - 1), K1 + 15) + base1
            plsc.store_scatter(i0_v, [pos0], idxv, mask=mk0)
            plsc.store_scatter(i1_v, [pos1], idxv, mask=mk1)
            c0v = c0v + plsc.all_reduce_population_count(mk0)
            c1v = c1v + plsc.all_reduce_population_count(mk1)
            return c0v, c1v

        c0v, c1v = plsc.parallel_loop(
            0, nchunk, 1, unroll=8, carry=(zc, zc))(scan_body)
        c0 = jnp.max(c0v)
        c1 = jnp.max(c1v)

        for (k, cN, base, iv, gv, rv) in (
                (K0, c0, base0, i0_v, g0_v, r0_v),
                (K1, c1, base1, i1_v, g1_v, r1_v)):
            cf = jnp.minimum(cN, k)
            fv = plsc.load_gather(iv, [jnp.full((16,), base, jnp.int32)])
            for t in range(k // 16):
                lane = iota + t * 16
                cur = iv[pl.ds(base + t * 16, 16)]
                cur = jnp.where(lane < cf, cur, fv)
                gv[pl.ds(base + t * 16, 16)] = cur + boff
                gx = plsc.load_gather(x_v, [cur]) - cxv
                gy = plsc.load_gather(y_v, [cur]) - cyv
                gz = plsc.load_gather(z_v, [cur]) - czv
                sidx = (iota * 3) + (base + t * 16) * 3
                plsc.store_scatter(rv, [sidx], gx)
                plsc.store_scatter(rv, [sidx + 1], gy)
                plsc.store_scatter(rv, [sidx + 2], gz)
        return carry_unused

    lax.fori_loop(0, _MW, per_centroid, jnp.int32(0))

    K0T, K1T = _MW * K0, _MW * K1
    for g in range(K0T // _GROWS):
        pltpu.async_copy(
            feat_h.at[g0_v.at[pl.ds(g * _GROWS, _GROWS)]], gf_v, sem).wait()
        pltpu.sync_copy(
            gf_v, gf0_h.at[pl.ds((b * NPOINT + m0) * K0 + g * _GROWS, _GROWS)])
    for g in range(K1T // _GROWS):
        pltpu.async_copy(
            feat_h.at[g1_v.at[pl.ds(g * _GROWS, _GROWS)]], gf_v, sem).wait()
        pltpu.sync_copy(
            gf_v, gf1_h.at[pl.ds((b * NPOINT + m0) * K1 + g * _GROWS, _GROWS)])
    pltpu.sync_copy(r0_v, rel0_h.at[pl.ds((b * NPOINT + m0) * K0 * 3, K0T * 3)])
    pltpu.sync_copy(r1_v, rel1_h.at[pl.ds((b * NPOINT + m0) * K1 * 3, K1T * 3)])


def _sc_ballquery_gather(xyzT, cxyzT, featflat):
    K0, K1 = NSAMPLES
    mesh = plsc.VectorSubcoreMesh(core_axis_name="c", subcore_axis_name="s")
    out_type = [
        jax.ShapeDtypeStruct((B * NPOINT * K0, IN_CH), jnp.float32),
        jax.ShapeDtypeStruct((B * NPOINT * K1, IN_CH), jnp.float32),
        jax.ShapeDtypeStruct((B * NPOINT * K0 * 3,), jnp.float32),
        jax.ShapeDtypeStruct((B * NPOINT * K1 * 3,), jnp.float32),
    ]
    scratch = [
        pltpu.VMEM((N,), jnp.float32),
        pltpu.VMEM((N,), jnp.float32),
        pltpu.VMEM((N,), jnp.float32),
        pltpu.VMEM((_MW,), jnp.float32),
        pltpu.VMEM((_MW,), jnp.float32),
        pltpu.VMEM((_MW,), jnp.float32),
        pltpu.VMEM((_MW * K0 + 16,), jnp.int32),
        pltpu.VMEM((_MW * K1 + 16,), jnp.int32),
        pltpu.VMEM((_MW * K0,), jnp.int32),
        pltpu.VMEM((_MW * K1,), jnp.int32),
        pltpu.VMEM((_MW * K0 * 3,), jnp.float32),
        pltpu.VMEM((_MW * K1 * 3,), jnp.float32),
        pltpu.VMEM((_GROWS, IN_CH), jnp.float32),
        pltpu.SemaphoreType.DMA,
    ]
    gf0, gf1, rel0, rel1 = pl.kernel(
        _sc_body, out_type=out_type, mesh=mesh, scratch_types=scratch,
        compiler_params=pltpu.CompilerParams(
            needs_layout_passes=False, use_tc_tiling_on_sc=False),
    )(xyzT.reshape(-1), cxyzT.reshape(-1), featflat)
    return gf0, gf1, rel0.reshape(-1, 3), rel1.reshape(-1, 3)


def _ball_query(radius, k, xyz, new_xyz):
    b, n, _ = xyz.shape
    d2 = jnp.sum((new_xyz[:, :, None, :] - xyz[:, None, :, :]) ** 2, axis=-1)
    mask = d2 < radius * radius
    key = jnp.where(mask, jnp.arange(n)[None, None, :], n)
    srt = jnp.sort(key, axis=-1)[..., :k]
    first = srt[..., :1]
    idx = jnp.where(srt >= n, first, srt)
    idx = jnp.where(idx >= n, 0, idx)
    return idx


def _bn(x, gamma, beta, axes):
    mu = jnp.mean(x, axis=axes, keepdims=True)
    var = jnp.var(x, axis=axes, keepdims=True)
    return (x - mu) / jnp.sqrt(var + 1e-5) * gamma + beta


_TR = 4096


def _mlp_y(gf_ref, rel_ref, wax_ref, waf_ref):
    gf = gf_ref[...]
    rel = rel_ref[...]
    y = jax.lax.dot_general(gf, waf_ref[...], (((1,), (0,)), ((), ())),
                            preferred_element_type=jnp.float32)
    y = y + rel[:, 0:1] * wax_ref[0:1, :]
    y = y + rel[:, 1:2] * wax_ref[1:2, :]
    y = y + rel[:, 2:3] * wax_ref[2:3, :]
    return y


def _p1_kernel(gf_ref, rel_ref, wax_ref, waf_ref, s_ref):
    y = _mlp_y(gf_ref, rel_ref, wax_ref, waf_ref)

    @pl.when(pl.program_id(0) == 0)
    def _():
        s_ref[...] = jnp.zeros_like(s_ref)

    s_ref[0:1, :] += jnp.sum(y, 0, keepdims=True)
    s_ref[1:2, :] += jnp.sum(y * y, 0, keepdims=True)


def _p2_kernel(gf_ref, rel_ref, wax_ref, waf_ref, a1_ref, c1_ref, wb_ref,
               s_ref):
    y = _mlp_y(gf_ref, rel_ref, wax_ref, waf_ref)
    h = jnp.maximum(y * a1_ref[...] + c1_ref[...], 0.0)
    z = jax.lax.dot_general(h, wb_ref[...], (((1,), (0,)), ((), ())),
                            preferred_element_type=jnp.float32)

    @pl.when(pl.program_id(0) == 0)
    def _():
        s_ref[...] = jnp.zeros_like(s_ref)

    s_ref[0:1, :] += jnp.sum(z, 0, keepdims=True)
    s_ref[1:2, :] += jnp.sum(z * z, 0, keepdims=True)


def _p3_kernel(k_const, gf_ref, rel_ref, wax_ref, waf_ref, a1_ref, c1_ref,
               wb_ref, a2_ref, c2_ref, o_ref):
    y = _mlp_y(gf_ref, rel_ref, wax_ref, waf_ref)
    h = jnp.maximum(y * a1_ref[...] + c1_ref[...], 0.0)
    z = jax.lax.dot_general(h, wb_ref[...], (((1,), (0,)), ((), ())),
                            preferred_element_type=jnp.float32)
    o = jnp.maximum(z * a2_ref[...] + c2_ref[...], 0.0)
    o_ref[...] = jnp.max(o.reshape(_TR // k_const, k_const, OUT), axis=1)


def _branch_mlp(gf, rel, k, Wa, ga, ba, Wb, gb, bb):
    R = gf.shape[0]
    ntiles = R // _TR
    wax = Wa[:, :3].T
    waf = Wa[:, 3:].T
    wbT = Wb.T
    half = Wa.shape[0]
    row_spec = [
        pl.BlockSpec((_TR, IN_CH), lambda i: (i, 0)),
        pl.BlockSpec((_TR, 3), lambda i: (i, 0)),
        pl.BlockSpec((3, half), lambda i: (0, 0)),
        pl.BlockSpec((IN_CH, half), lambda i: (0, 0)),
    ]
    acc_spec = pl.BlockSpec((2, half), lambda i: (0, 0))
    s1 = pl.pallas_call(
        _p1_kernel, grid=(ntiles,), in_specs=row_spec, out_specs=acc_spec,
        out_shape=jax.ShapeDtypeStruct((2, half), jnp.float32),
    )(gf, rel, wax, waf)
    mu1 = s1[0] / R
    var1 = s1[1] / R - mu1 * mu1
    a1 = (ga / jnp.sqrt(var1 + 1e-5)).reshape(1, half)
    c1 = (ba - mu1 * ga / jnp.sqrt(var1 + 1e-5)).reshape(1, half)

    vec_specs = [
        pl.BlockSpec((1, half), lambda i: (0, 0)),
        pl.BlockSpec((1, half), lambda i: (0, 0)),
        pl.BlockSpec((half, OUT), lambda i: (0, 0)),
    ]
    acc2_spec = pl.BlockSpec((2, OUT), lambda i: (0, 0))
    s2 = pl.pallas_call(
        _p2_kernel, grid=(ntiles,), in_specs=row_spec + vec_specs,
        out_specs=acc2_spec,
        out_shape=jax.ShapeDtypeStruct((2, OUT), jnp.float32),
    )(gf, rel, wax, waf, a1, c1, wbT)
    mu2 = s2[0] / R
    var2 = s2[1] / R - mu2 * mu2
    a2 = (gb / jnp.sqrt(var2 + 1e-5)).reshape(1, OUT)
    c2 = (bb - mu2 * gb / jnp.sqrt(var2 + 1e-5)).reshape(1, OUT)

    vec2_specs = [
        pl.BlockSpec((1, OUT), lambda i: (0, 0)),
        pl.BlockSpec((1, OUT), lambda i: (0, 0)),
    ]
    pooled = pl.pallas_call(
        functools.partial(_p3_kernel, k),
        grid=(ntiles,), in_specs=row_spec + vec_specs + vec2_specs,
        out_specs=pl.BlockSpec((_TR // k, OUT), lambda i: (i, 0)),
        out_shape=jax.ShapeDtypeStruct((R // k, OUT), jnp.float32),
    )(gf, rel, wax, waf, a1, c1, wbT, a2, c2)
    return pooled.reshape(B, NPOINT, OUT)


def _final_mlp_kernel(x_ref, wf1_ref, gf1_ref, bf1_ref, wf2_ref, gf2_ref, bf2_ref, o_ref):
    x = x_ref[...]
    y = jax.lax.dot_general(x, wf1_ref[...], (((1,), (1,)), ((), ())),
                            preferred_element_type=jnp.float32)
    mu = jnp.mean(y, axis=0, keepdims=True)
    var = jnp.mean((y - mu) ** 2, axis=0, keepdims=True)
    y = (y - mu) / jnp.sqrt(var + 1e-5) * gf1_ref[...] + bf1_ref[...]
    y = jnp.maximum(y, 0.0)
    z = jax.lax.dot_general(y, wf2_ref[...], (((1,), (1,)), ((), ())),
                            preferred_element_type=jnp.float32)
    mu = jnp.mean(z, axis=0, keepdims=True)
    var = jnp.mean((z - mu) ** 2, axis=0, keepdims=True)
    z = (z - mu) / jnp.sqrt(var + 1e-5) * gf2_ref[...] + bf2_ref[...]
    o_ref[...] = jnp.maximum(z, 0.0)


def _final_mlp(x, Wf1, gf1, bf1, Wf2, gf2, bf2):
    bm = x.shape[0] * x.shape[1]
    out = pl.pallas_call(
        _final_mlp_kernel,
        out_shape=jax.ShapeDtypeStruct((bm, OUT), jnp.float32),
    )(x.reshape(bm, -1), Wf1, gf1.reshape(1, OUT), bf1.reshape(1, OUT),
      Wf2, gf2.reshape(1, OUT), bf2.reshape(1, OUT))
    return out.reshape(x.shape[0], x.shape[1], OUT)


def kernel(xyz, feat, W0_1, g0_1, b0_1, W0_2, g0_2, b0_2, W1_1, g1_1, b1_1,
           W1_2, g1_2, b1_2, Wf1, gf1, bf1, Wf2, gf2, bf2):
    new_xyz, cxyzT, xyzT = _fps_newxyz(xyz)
    featflat = feat.reshape(B * N, IN_CH)
    gfa, gfb, rela, relb = _sc_ballquery_gather(xyzT, cxyzT, featflat)
    pooled0 = _branch_mlp(gfa, rela, NSAMPLES[0], W0_1, g0_1, b0_1, W0_2, g0_2, b0_2)
    pooled1 = _branch_mlp(gfb, relb, NSAMPLES[1], W1_1, g1_1, b1_1, W1_2, g1_2, b1_2)
    x = jnp.concatenate([pooled0, pooled1], axis=-1)
    x = _final_mlp(x, Wf1, gf1, bf1, Wf2, gf2, bf2)
    return (new_xyz, x)


# speedup vs baseline: 1.2814x; 1.0889x over previous
import functools

import jax
import jax.numpy as jnp
from jax import lax
from jax.experimental import pallas as pl
from jax.experimental.pallas import tpu as pltpu
from jax.experimental.pallas import tpu_sc as plsc

B, N, NPOINT = 4, 8192, 1024
RADII = (0.1, 0.2)
NSAMPLES = (32, 64)
IN_CH = 64
OUT = 128


def _fps_kernel(xyz_ref, o_ref):
    rows, cols = 8, N // 8
    idx2d = (jax.lax.broadcasted_iota(jnp.int32, (rows, cols), 0) * cols
             + jax.lax.broadcasted_iota(jnp.int32, (rows, cols), 1))
    ocols = NPOINT // 8
    t2d = (jax.lax.broadcasted_iota(jnp.int32, (8, ocols), 0) * ocols
           + jax.lax.broadcasted_iota(jnp.int32, (8, ocols), 1))

    def extract(sel2, arr):
        return jnp.max(jnp.where(sel2, arr, -1.0))

    init = []
    sel0 = idx2d == 0
    hit0 = t2d == 0
    zeros = jnp.zeros((8, ocols), jnp.float32)
    for b in range(B):
        cx = extract(sel0, xyz_ref[b, 0])
        cy = extract(sel0, xyz_ref[b, 1])
        cz = extract(sel0, xyz_ref[b, 2])
        init.append((jnp.full((rows, cols), 1e10, jnp.float32), cx, cy, cz,
                     jnp.where(hit0, cx, zeros), jnp.where(hit0, cy, zeros),
                     jnp.where(hit0, cz, zeros)))

    def body(t, carry):
        hit = t2d == t
        out = []
        for b in range(B):
            dists, cx, cy, cz, newx, newy, newz = carry[b]
            x = xyz_ref[b, 0]
            y = xyz_ref[b, 1]
            z = xyz_ref[b, 2]
            dx = x - cx
            dy = y - cy
            dz = z - cz
            d = dx * dx + dy * dy + dz * dz
            dists = jnp.minimum(dists, d)
            maxv = jnp.max(dists)
            fidx = jnp.min(jnp.where(dists == maxv, idx2d, N))
            sel2 = idx2d == fidx
            cx = extract(sel2, x)
            cy = extract(sel2, y)
            cz = extract(sel2, z)
            out.append((dists, cx, cy, cz, jnp.where(hit, cx, newx),
                        jnp.where(hit, cy, newy), jnp.where(hit, cz, newz)))
        return tuple(out)

    carry = jax.lax.fori_loop(1, NPOINT, body, tuple(init))
    for b in range(B):
        _, _, _, _, newx, newy, newz = carry[b]
        o_ref[b, 0] = newx
        o_ref[b, 1] = newy
        o_ref[b, 2] = newz


def _fps_newxyz(xyz):
    b, n, _ = xyz.shape
    xyzT = jnp.transpose(xyz, (0, 2, 1))
    xyzR = xyzT.reshape(b, 3, 8, n // 8)
    outR = pl.pallas_call(
        _fps_kernel,
        out_shape=jax.ShapeDtypeStruct((b, 3, 8, NPOINT // 8), jnp.float32),
    )(xyzR)
    cxyzT = outR.reshape(b, 3, NPOINT)
    return jnp.transpose(cxyzT, (0, 2, 1)), cxyzT, xyzT


_NW = 32
_MW = NPOINT // 8
_GROWS = 256


def _sc_body(xyzT_h, cxyzT_h, feat_h, gf0_h, gf1_h, rel0_h, rel1_h,
             x_v, y_v, z_v, cx_v, cy_v, cz_v, i0_v, i1_v, g0_v, g1_v,
             r0_v, r1_v, gf_v, sem):
    K0, K1 = NSAMPLES
    wid = lax.axis_index("s") * 2 + lax.axis_index("c")
    b = wid // 8
    m0 = (wid % 8) * _MW
    pltpu.sync_copy(xyzT_h.at[pl.ds((b * 3 + 0) * N, N)], x_v)
    pltpu.sync_copy(xyzT_h.at[pl.ds((b * 3 + 1) * N, N)], y_v)
    pltpu.sync_copy(xyzT_h.at[pl.ds((b * 3 + 2) * N, N)], z_v)
    pltpu.sync_copy(cxyzT_h.at[pl.ds((b * 3 + 0) * NPOINT + m0, _MW)], cx_v)
    pltpu.sync_copy(cxyzT_h.at[pl.ds((b * 3 + 1) * NPOINT + m0, _MW)], cy_v)
    pltpu.sync_copy(cxyzT_h.at[pl.ds((b * 3 + 2) * NPOINT + m0, _MW)], cz_v)
    iota = lax.iota(jnp.int32, 16)
    r2a = jnp.float32(RADII[0] * RADII[0])
    r2b = jnp.float32(RADII[1] * RADII[1])
    nchunk = N // 16
    boff = b * N

    nsup = nchunk // 16

    def per_centroid(m, carry_unused):
        mm = jnp.full((16,), m, jnp.int32)
        cxv = plsc.load_gather(cx_v, [mm])
        cyv = plsc.load_gather(cy_v, [mm])
        czv = plsc.load_gather(cz_v, [mm])
        base0 = m * K0
        base1 = m * K1
        zc = jnp.zeros((16,), jnp.int32)

        def scan_body(j, cnts):
            c0v, c1v = cnts
            off = j * 16
            dx = x_v[pl.ds(off, 16)] - cxv
            dy = y_v[pl.ds(off, 16)] - cyv
            dz = z_v[pl.ds(off, 16)] - czv
            d2 = dx * dx + dy * dy + dz * dz
            mk0 = d2 < r2a
            mk1 = d2 < r2b
            idxv = iota + off
            rk0 = plsc.cumsum(mk0.astype(jnp.int32))
            rk1 = plsc.cumsum(mk1.astype(jnp.int32))
            pos0 = jnp.minimum(c0v + (rk0 - 1), K0 + 15) + base0
            pos1 = jnp.minimum(c1v + (rk1 - 1), K1 + 15) + base1
            plsc.store_scatter(i0_v, [pos0], idxv, mask=mk0)
            plsc.store_scatter(i1_v, [pos1], idxv, mask=mk1)
            c0v = c0v + plsc.all_reduce_population_count(mk0)
            c1v = c1v + plsc.all_reduce_population_count(mk1)
            return c0v, c1v

        c0v, c1v = plsc.parallel_loop(
            0, nchunk, 1, unroll=4, carry=(zc, zc))(scan_body)
        c0 = jnp.max(c0v)
        c1 = jnp.max(c1v)

        for (k, cN, base, iv, gv, rv) in (
                (K0, c0, base0, i0_v, g0_v, r0_v),
                (K1, c1, base1, i1_v, g1_v, r1_v)):
            cf = jnp.minimum(cN, k)
            fv = plsc.load_gather(iv, [jnp.full((16,), base, jnp.int32)])
            for t in range(k // 16):
                lane = iota + t * 16
                cur = iv[pl.ds(base + t * 16, 16)]
                cur = jnp.where(lane < cf, cur, fv)
                gv[pl.ds(base + t * 16, 16)] = cur + boff
                gx = plsc.load_gather(x_v, [cur]) - cxv
                gy = plsc.load_gather(y_v, [cur]) - cyv
                gz = plsc.load_gather(z_v, [cur]) - czv
                sidx = (iota * 3) + (base + t * 16) * 3
                plsc.store_scatter(rv, [sidx], gx)
                plsc.store_scatter(rv, [sidx + 1], gy)
                plsc.store_scatter(rv, [sidx + 2], gz)
        return carry_unused

    lax.fori_loop(0, _MW, per_centroid, jnp.int32(0))

    K0T, K1T = _MW * K0, _MW * K1
    for g in range(K0T // _GROWS):
        pltpu.async_copy(
            feat_h.at[g0_v.at[pl.ds(g * _GROWS, _GROWS)]], gf_v, sem).wait()
        pltpu.sync_copy(
            gf_v, gf0_h.at[pl.ds((b * NPOINT + m0) * K0 + g * _GROWS, _GROWS)])
    for g in range(K1T // _GROWS):
        pltpu.async_copy(
            feat_h.at[g1_v.at[pl.ds(g * _GROWS, _GROWS)]], gf_v, sem).wait()
        pltpu.sync_copy(
            gf_v, gf1_h.at[pl.ds((b * NPOINT + m0) * K1 + g * _GROWS, _GROWS)])
    pltpu.sync_copy(r0_v, rel0_h.at[pl.ds((b * NPOINT + m0) * K0 * 3, K0T * 3)])
    pltpu.sync_copy(r1_v, rel1_h.at[pl.ds((b * NPOINT + m0) * K1 * 3, K1T * 3)])


def _sc_ballquery_gather(xyzT, cxyzT, featflat):
    K0, K1 = NSAMPLES
    mesh = plsc.VectorSubcoreMesh(core_axis_name="c", subcore_axis_name="s")
    out_type = [
        jax.ShapeDtypeStruct((B * NPOINT * K0, IN_CH), jnp.float32),
        jax.ShapeDtypeStruct((B * NPOINT * K1, IN_CH), jnp.float32),
        jax.ShapeDtypeStruct((B * NPOINT * K0 * 3,), jnp.float32),
        jax.ShapeDtypeStruct((B * NPOINT * K1 * 3,), jnp.float32),
    ]
    scratch = [
        pltpu.VMEM((N,), jnp.float32),
        pltpu.VMEM((N,), jnp.float32),
        pltpu.VMEM((N,), jnp.float32),
        pltpu.VMEM((_MW,), jnp.float32),
        pltpu.VMEM((_MW,), jnp.float32),
        pltpu.VMEM((_MW,), jnp.float32),
        pltpu.VMEM((_MW * K0 + 16,), jnp.int32),
        pltpu.VMEM((_MW * K1 + 16,), jnp.int32),
        pltpu.VMEM((_MW * K0,), jnp.int32),
        pltpu.VMEM((_MW * K1,), jnp.int32),
        pltpu.VMEM((_MW * K0 * 3,), jnp.float32),
        pltpu.VMEM((_MW * K1 * 3,), jnp.float32),
        pltpu.VMEM((_GROWS, IN_CH), jnp.float32),
        pltpu.SemaphoreType.DMA,
    ]
    gf0, gf1, rel0, rel1 = pl.kernel(
        _sc_body, out_type=out_type, mesh=mesh, scratch_types=scratch,
        compiler_params=pltpu.CompilerParams(
            needs_layout_passes=False, use_tc_tiling_on_sc=False),
    )(xyzT.reshape(-1), cxyzT.reshape(-1), featflat)
    return gf0, gf1, rel0.reshape(-1, 3), rel1.reshape(-1, 3)


def _ball_query(radius, k, xyz, new_xyz):
    b, n, _ = xyz.shape
    d2 = jnp.sum((new_xyz[:, :, None, :] - xyz[:, None, :, :]) ** 2, axis=-1)
    mask = d2 < radius * radius
    key = jnp.where(mask, jnp.arange(n)[None, None, :], n)
    srt = jnp.sort(key, axis=-1)[..., :k]
    first = srt[..., :1]
    idx = jnp.where(srt >= n, first, srt)
    idx = jnp.where(idx >= n, 0, idx)
    return idx


def _bn(x, gamma, beta, axes):
    mu = jnp.mean(x, axis=axes, keepdims=True)
    var = jnp.var(x, axis=axes, keepdims=True)
    return (x - mu) / jnp.sqrt(var + 1e-5) * gamma + beta


_TR = 4096


def _mlp_y(gf_ref, rel_ref, wax_ref, waf_ref):
    gf = gf_ref[...]
    rel = rel_ref[...]
    y = jax.lax.dot_general(gf, waf_ref[...], (((1,), (0,)), ((), ())),
                            preferred_element_type=jnp.float32)
    y = y + rel[:, 0:1] * wax_ref[0:1, :]
    y = y + rel[:, 1:2] * wax_ref[1:2, :]
    y = y + rel[:, 2:3] * wax_ref[2:3, :]
    return y


def _p1_kernel(gf_ref, rel_ref, wax_ref, waf_ref, s_ref):
    y = _mlp_y(gf_ref, rel_ref, wax_ref, waf_ref)

    @pl.when(pl.program_id(0) == 0)
    def _():
        s_ref[...] = jnp.zeros_like(s_ref)

    s_ref[0:1, :] += jnp.sum(y, 0, keepdims=True)
    s_ref[1:2, :] += jnp.sum(y * y, 0, keepdims=True)


def _p2_kernel(gf_ref, rel_ref, wax_ref, waf_ref, a1_ref, c1_ref, wb_ref,
               s_ref):
    y = _mlp_y(gf_ref, rel_ref, wax_ref, waf_ref)
    h = jnp.maximum(y * a1_ref[...] + c1_ref[...], 0.0)
    z = jax.lax.dot_general(h, wb_ref[...], (((1,), (0,)), ((), ())),
                            preferred_element_type=jnp.float32)

    @pl.when(pl.program_id(0) == 0)
    def _():
        s_ref[...] = jnp.zeros_like(s_ref)

    s_ref[0:1, :] += jnp.sum(z, 0, keepdims=True)
    s_ref[1:2, :] += jnp.sum(z * z, 0, keepdims=True)


def _p3_kernel(k_const, gf_ref, rel_ref, wax_ref, waf_ref, a1_ref, c1_ref,
               wb_ref, a2_ref, c2_ref, o_ref):
    y = _mlp_y(gf_ref, rel_ref, wax_ref, waf_ref)
    h = jnp.maximum(y * a1_ref[...] + c1_ref[...], 0.0)
    z = jax.lax.dot_general(h, wb_ref[...], (((1,), (0,)), ((), ())),
                            preferred_element_type=jnp.float32)
    o = jnp.maximum(z * a2_ref[...] + c2_ref[...], 0.0)
    o_ref[...] = jnp.max(o.reshape(_TR // k_const, k_const, OUT), axis=1)


def _branch_mlp(gf, rel, k, Wa, ga, ba, Wb, gb, bb):
    R = gf.shape[0]
    ntiles = R // _TR
    wax = Wa[:, :3].T
    waf = Wa[:, 3:].T
    wbT = Wb.T
    half = Wa.shape[0]
    row_spec = [
        pl.BlockSpec((_TR, IN_CH), lambda i: (i, 0)),
        pl.BlockSpec((_TR, 3), lambda i: (i, 0)),
        pl.BlockSpec((3, half), lambda i: (0, 0)),
        pl.BlockSpec((IN_CH, half), lambda i: (0, 0)),
    ]
    acc_spec = pl.BlockSpec((2, half), lambda i: (0, 0))
    s1 = pl.pallas_call(
        _p1_kernel, grid=(ntiles,), in_specs=row_spec, out_specs=acc_spec,
        out_shape=jax.ShapeDtypeStruct((2, half), jnp.float32),
    )(gf, rel, wax, waf)
    mu1 = s1[0] / R
    var1 = s1[1] / R - mu1 * mu1
    a1 = (ga / jnp.sqrt(var1 + 1e-5)).reshape(1, half)
    c1 = (ba - mu1 * ga / jnp.sqrt(var1 + 1e-5)).reshape(1, half)

    vec_specs = [
        pl.BlockSpec((1, half), lambda i: (0, 0)),
        pl.BlockSpec((1, half), lambda i: (0, 0)),
        pl.BlockSpec((half, OUT), lambda i: (0, 0)),
    ]
    acc2_spec = pl.BlockSpec((2, OUT), lambda i: (0, 0))
    s2 = pl.pallas_call(
        _p2_kernel, grid=(ntiles,), in_specs=row_spec + vec_specs,
        out_specs=acc2_spec,
        out_shape=jax.ShapeDtypeStruct((2, OUT), jnp.float32),
    )(gf, rel, wax, waf, a1, c1, wbT)
    mu2 = s2[0] / R
    var2 = s2[1] / R - mu2 * mu2
    a2 = (gb / jnp.sqrt(var2 + 1e-5)).reshape(1, OUT)
    c2 = (bb - mu2 * gb / jnp.sqrt(var2 + 1e-5)).reshape(1, OUT)

    vec2_specs = [
        pl.BlockSpec((1, OUT), lambda i: (0, 0)),
        pl.BlockSpec((1, OUT), lambda i: (0, 0)),
    ]
    pooled = pl.pallas_call(
        functools.partial(_p3_kernel, k),
        grid=(ntiles,), in_specs=row_spec + vec_specs + vec2_specs,
        out_specs=pl.BlockSpec((_TR // k, OUT), lambda i: (i, 0)),
        out_shape=jax.ShapeDtypeStruct((R // k, OUT), jnp.float32),
    )(gf, rel, wax, waf, a1, c1, wbT, a2, c2)
    return pooled.reshape(B, NPOINT, OUT)


def _final_mlp_kernel(x_ref, wf1_ref, gf1_ref, bf1_ref, wf2_ref, gf2_ref, bf2_ref, o_ref):
    x = x_ref[...]
    y = jax.lax.dot_general(x, wf1_ref[...], (((1,), (1,)), ((), ())),
                            preferred_element_type=jnp.float32)
    mu = jnp.mean(y, axis=0, keepdims=True)
    var = jnp.mean((y - mu) ** 2, axis=0, keepdims=True)
    y = (y - mu) / jnp.sqrt(var + 1e-5) * gf1_ref[...] + bf1_ref[...]
    y = jnp.maximum(y, 0.0)
    z = jax.lax.dot_general(y, wf2_ref[...], (((1,), (1,)), ((), ())),
                            preferred_element_type=jnp.float32)
    mu = jnp.mean(z, axis=0, keepdims=True)
    var = jnp.mean((z - mu) ** 2, axis=0, keepdims=True)
    z = (z - mu) / jnp.sqrt(var + 1e-5) * gf2_ref[...] + bf2_ref[...]
    o_ref[...] = jnp.maximum(z, 0.0)


def _final_mlp(x, Wf1, gf1, bf1, Wf2, gf2, bf2):
    bm = x.shape[0] * x.shape[1]
    out = pl.pallas_call(
        _final_mlp_kernel,
        out_shape=jax.ShapeDtypeStruct((bm, OUT), jnp.float32),
    )(x.reshape(bm, -1), Wf1, gf1.reshape(1, OUT), bf1.reshape(1, OUT),
      Wf2, gf2.reshape(1, OUT), bf2.reshape(1, OUT))
    return out.reshape(x.shape[0], x.shape[1], OUT)


def kernel(xyz, feat, W0_1, g0_1, b0_1, W0_2, g0_2, b0_2, W1_1, g1_1, b1_1,
           W1_2, g1_2, b1_2, Wf1, gf1, bf1, Wf2, gf2, bf2):
    new_xyz, cxyzT, xyzT = _fps_newxyz(xyz)
    featflat = feat.reshape(B * N, IN_CH)
    gfa, gfb, rela, relb = _sc_ballquery_gather(xyzT, cxyzT, featflat)
    pooled0 = _branch_mlp(gfa, rela, NSAMPLES[0], W0_1, g0_1, b0_1, W0_2, g0_2, b0_2)
    pooled1 = _branch_mlp(gfb, relb, NSAMPLES[1], W1_1, g1_1, b1_1, W1_2, g1_2, b1_2)
    x = jnp.concatenate([pooled0, pooled1], axis=-1)
    x = _final_mlp(x, Wf1, gf1, bf1, Wf2, gf2, bf2)
    return (new_xyz, x)
